# baseline (speedup 1.0000x reference)
import jax
import jax.numpy as jnp
from jax.experimental import pallas as pl
from jax.experimental.pallas import tpu as pltpu

NUM_CLASSES = 80
REG_MAX = 7
STRIDES = [8, 16, 32, 64]
FEAT_SHAPES = [(64, 64), (32, 32), (16, 16), (8, 8)]
CENTER_RADIUS = 2.5
CAND_TOPK = 10
IOU_W = 3.0
CLS_W = 1.0
VFL_ALPHA = 0.75
VFL_GAMMA = 2.0
W_VFL = 1.0
W_GIOU = 2.0
W_DFL = 0.25

N_PRIORS = sum(h * w for h, w in FEAT_SHAPES)
BIGI = 10**9


def _priors():
    out = []
    for (h, w), s in zip(FEAT_SHAPES, STRIDES):
        ys = (jnp.arange(h, dtype=jnp.float32) + 0.5) * s
        xs = (jnp.arange(w, dtype=jnp.float32) + 0.5) * s
        yy, xx = jnp.meshgrid(ys, xs, indexing='ij')
        cx = xx.reshape(-1)
        cy = yy.reshape(-1)
        ss = jnp.full_like(cx, float(s))
        out.append(jnp.stack([cx, cy, ss, ss], axis=-1))
    return jnp.concatenate(out, axis=0)


def _body(cls_ref, reg_ref, gtb_ref, gtl_ref, pri_ref, out_ref):
    N = N_PRIORS
    M = gtl_ref.shape[1]
    f32 = jnp.float32

    cls = cls_ref[0]
    reg = reg_ref[0]
    gtb = gtb_ref[0]
    gtl = gtl_ref[0]
    cx = pri_ref[0:1, :]
    cy = pri_ref[1:2, :]
    st = pri_ref[2:3, :]

    colmax = jnp.max(reg, axis=0, keepdims=True)
    e_in = reg - colmax
    ex = jnp.exp(e_in)
    sub32 = jax.lax.broadcasted_iota(jnp.int32, (32, N), 0)
    grp = sub32 // 8
    j8 = (sub32 % 8).astype(f32)
    dists = []
    ldens = []
    for g in range(4):
        m = grp == g
        den = jnp.sum(jnp.where(m, ex, 0.0), axis=0, keepdims=True)
        num = jnp.sum(jnp.where(m, ex * j8, 0.0), axis=0, keepdims=True)
        dists.append(num / den * st)
        ldens.append(jnp.log(den))
    x1 = cx - dists[0]
    y1 = cy - dists[1]
    x2 = cx + dists[2]
    y2 = cy + dists[3]

    ecls = jnp.exp(cls)
    p = ecls / (1.0 + ecls)
    sp = jnp.log(1.0 + ecls)
    l1 = jnp.maximum(cls - sp, jnp.log(jnp.float32(1e-12)))
    l0 = jnp.maximum(-sp, jnp.log(jnp.float32(1e-12)))
    log_lo = jnp.log(jnp.float32(1e-7))
    log_hi = jnp.log(jnp.float32(1.0 - 1e-7))
    l1c = jnp.clip(l1, log_lo, log_hi)
    l0c = jnp.clip(l0, log_lo, log_hi)
    S = -jnp.sum(l0c, axis=0, keepdims=True)
    Acost = l0c - l1c
    oh_mc = (jax.lax.broadcasted_iota(jnp.int32, (M, NUM_CLASSES), 1)
             == gtl).astype(f32)
    gsel = jax.lax.dot_general(oh_mc, Acost, (((1,), (0,)), ((), ())),
                               preferred_element_type=f32,
                               precision=jax.lax.Precision.HIGHEST)
    cls_cost = S + gsel

    gx1 = gtb[:, 0:1]
    gy1 = gtb[:, 1:2]
    gx2 = gtb[:, 2:3]
    gy2 = gtb[:, 3:4]
    l_ = cx - gx1
    t_ = cy - gy1
    r_ = gx2 - cx
    b_ = gy2 - cy
    gmin = jnp.minimum(jnp.minimum(l_, t_), jnp.minimum(r_, b_))
    gcx = (gx1 + gx2) * 0.5
    gcy = (gy1 + gy2) * 0.5
    cl_ = cx - (gcx - CENTER_RADIUS * st)
    ct_ = cy - (gcy - CENTER_RADIUS * st)
    cr_ = (gcx + CENTER_RADIUS * st) - cx
    cb_ = (gcy + CENTER_RADIUS * st) - cy
    cmin = jnp.minimum(jnp.minimum(cl_, ct_), jnp.minimum(cr_, cb_))
    bothv = jnp.minimum(gmin, cmin)
    validv = jnp.max(jnp.maximum(gmin, cmin), axis=0, keepdims=True)
    validf = (validv > 0).astype(f32)

    area_p = (x2 - x1) * (y2 - y1)
    area_g = (gx2 - gx1) * (gy2 - gy1)
    tlx = jnp.maximum(x1, gx1)
    tly = jnp.maximum(y1, gy1)
    brx = jnp.minimum(x2, gx2)
    bry = jnp.minimum(y2, gy2)
    iw = jnp.clip(brx - tlx, 0.0, None)
    ih = jnp.clip(bry - tly, 0.0, None)
    inter = iw * ih
    ious = inter / (area_p + area_g - inter + 1e-16) * validf
    iou_cost = -jnp.log(ious + 1e-8)
    cost = (CLS_W * cls_cost + IOU_W * iou_cost
            + jnp.where(bothv > 0, 0.0, 1e5) + 1e5 * (1.0 - validf))

    iotaL = jax.lax.broadcasted_iota(jnp.int32, (M, N), 1)
    iotaS = jax.lax.broadcasted_iota(jnp.int32, (M, N), 0)

    iou_work = ious
    topk_sum = jnp.zeros((M, 1), f32)
    for _ in range(CAND_TOPK):
        mval = jnp.max(iou_work, axis=1, keepdims=True)
        topk_sum = topk_sum + mval
        idx = jnp.argmax(iou_work, axis=1, keepdims=True)
        iou_work = jnp.where(iotaL == idx, -1.0, iou_work)
    dyn_k = jnp.maximum(topk_sum.astype(jnp.int32), 1)

    cost_work = cost
    matchf = jnp.zeros((M, N), f32)
    for j in range(CAND_TOPK):
        idx = jnp.argmin(cost_work, axis=1, keepdims=True)
        jltf = (j < dyn_k).astype(f32)
        hit = iotaL == idx
        matchf = jnp.where(hit, jltf, matchf)
        cost_work = jnp.where(hit, 1e30, cost_work)
    matchingf = matchf * validf

    cnt = jnp.sum(matchingf, axis=0, keepdims=True)
    masked_cost = jnp.where(matchingf > 0, cost, 1e30)
    mgt = jnp.argmin(masked_cost, axis=0, keepdims=True)
    fg = (cnt > 0).astype(f32)
    oh_m = iotaS == mgt
    miou = jnp.sum(jnp.where(oh_m, ious, 0.0), axis=0, keepdims=True) * fg

    vfl_bg = jnp.sum((VFL_ALPHA * (p * p)) * (-l0))
    dnums = (((1,), (0,)), ((), ()))
    PgM = jax.lax.dot_general(oh_mc, p, dnums, preferred_element_type=f32,
                              precision=jax.lax.Precision.HIGHEST)
    L1gM = jax.lax.dot_general(oh_mc, l1, dnums, preferred_element_type=f32,
                               precision=jax.lax.Precision.HIGHEST)
    L0gM = jax.lax.dot_general(oh_mc, l0, dnums, preferred_element_type=f32,
                               precision=jax.lax.Precision.HIGHEST)
    p_g = jnp.sum(jnp.where(oh_m, PgM, 0.0), axis=0, keepdims=True)
    l1_g = jnp.sum(jnp.where(oh_m, L1gM, 0.0), axis=0, keepdims=True)
    l0_g = jnp.sum(jnp.where(oh_m, L0gM, 0.0), axis=0, keepdims=True)
    qn = miou * fg
    bce_fg = -(qn * l1_g + (1.0 - qn) * l0_g)
    fw_fg = jnp.where(qn > 0, qn, VFL_ALPHA * p_g * p_g)
    bg_at = VFL_ALPHA * p_g * p_g * (-l0_g)
    vfl = vfl_bg + jnp.sum(bce_fg * fw_fg - bg_at)

    tx1 = jnp.sum(jnp.where(oh_m, gx1, 0.0), axis=0, keepdims=True)
    ty1 = jnp.sum(jnp.where(oh_m, gy1, 0.0), axis=0, keepdims=True)
    tx2 = jnp.sum(jnp.where(oh_m, gx2, 0.0), axis=0, keepdims=True)
    ty2 = jnp.sum(jnp.where(oh_m, gy2, 0.0), axis=0, keepdims=True)
    w = miou * fg
    ttlx = jnp.maximum(x1, tx1)
    ttly = jnp.maximum(y1, ty1)
    tbrx = jnp.minimum(x2, tx2)
    tbry = jnp.minimum(y2, ty2)
    giw = jnp.clip(tbrx - ttlx, 0.0, None)
    gih = jnp.clip(tbry - ttly, 0.0, None)
    ginter = giw * gih
    ap = jnp.clip(x2 - x1, 0.0, None) * jnp.clip(y2 - y1, 0.0, None)
    at = jnp.clip(tx2 - tx1, 0.0, None) * jnp.clip(ty2 - ty1, 0.0, None)
    union = ap + at - ginter
    giou_iou = ginter / (union + 1e-16)
    ctlx = jnp.minimum(x1, tx1)
    ctly = jnp.minimum(y1, ty1)
    cbrx = jnp.maximum(x2, tx2)
    cbry = jnp.maximum(y2, ty2)
    cw = jnp.clip(cbrx - ctlx, 0.0, None)
    ch = jnp.clip(cbry - ctly, 0.0, None)
    ac = cw * ch
    giou_l = 1.0 - (giou_iou - (ac - union) / (ac + 1e-16))
    gl = jnp.sum(giou_l * w)

    left = (cx - tx1) / st
    top = (cy - ty1) / st
    right = (tx2 - cx) / st
    bot = (ty2 - cy) / st
    td = jnp.where(grp == 0, left,
                   jnp.where(grp == 1, top,
                             jnp.where(grp == 2, right, bot)))
    td = jnp.clip(td, 0.0, REG_MAX - 0.1)
    dlf = jnp.floor(td)
    drf = jnp.minimum(dlf + 1.0, float(REG_MAX))
    wl = dlf + 1.0 - td
    wr = td - dlf
    ld = jnp.where(grp == 0, ldens[0],
                   jnp.where(grp == 1, ldens[1],
                             jnp.where(grp == 2, ldens[2], ldens[3])))
    logp = e_in - ld
    Wt = wl * (j8 == dlf).astype(f32) + wr * (j8 == drf).astype(f32)
    ce_sum = jnp.sum(-logp * Wt, axis=0, keepdims=True)
    dfl = jnp.sum((ce_sum / 4.0) * w)

    num_pos = jnp.sum(fg)
    wsum = jnp.sum(w)

    lane = jax.lax.broadcasted_iota(jnp.int32, (1, 128), 1)
    row = (jnp.where(lane == 0, vfl, 0.0) + jnp.where(lane == 1, gl, 0.0)
           + jnp.where(lane == 2, dfl, 0.0) + jnp.where(lane == 3, num_pos, 0.0)
           + jnp.where(lane == 4, wsum, 0.0))
    out_ref[0] = row


@jax.jit
def kernel(cls_scores, reg_preds, gt_bboxes, gt_labels):
    B, N, C = cls_scores.shape
    M = gt_labels.shape[1]
    pri_t = jnp.transpose(_priors())
    cls_t = jnp.transpose(cls_scores, (0, 2, 1))
    reg_t = jnp.transpose(reg_preds, (0, 2, 1))
    gtl3 = gt_labels.reshape(B, M, 1).astype(jnp.int32)
    comps = pl.pallas_call(
        _body,
        grid=(B,),
        in_specs=[
            pl.BlockSpec((1, C, N), lambda b: (b, 0, 0)),
            pl.BlockSpec((1, 4 * (REG_MAX + 1), N), lambda b: (b, 0, 0)),
            pl.BlockSpec((1, M, 4), lambda b: (b, 0, 0)),
            pl.BlockSpec((1, M, 1), lambda b: (b, 0, 0)),
            pl.BlockSpec((4, N), lambda b: (0, 0)),
        ],
        out_specs=pl.BlockSpec((1, 1, 128), lambda b: (b, 0, 0)),
        out_shape=jax.ShapeDtypeStruct((B, 1, 128), jnp.float32),
        compiler_params=pltpu.CompilerParams(
            vmem_limit_bytes=100 * 1024 * 1024),
    )(cls_t, reg_t, gt_bboxes, gtl3, pri_t)
    vfl = comps[:, 0, 0]
    gl = comps[:, 0, 1]
    dfl = comps[:, 0, 2]
    num_pos = jnp.maximum(comps[:, 0, 3], 1.0)
    wsum = jnp.maximum(comps[:, 0, 4], 1.0)
    per_b = W_VFL * vfl / num_pos + W_GIOU * gl / wsum + W_DFL * dfl / wsum
    return jnp.mean(per_b)


# speedup vs baseline: 1.1058x; 1.1058x over previous
import jax
import jax.numpy as jnp
from jax.experimental import pallas as pl
from jax.experimental.pallas import tpu as pltpu

NUM_CLASSES = 80
REG_MAX = 7
STRIDES = [8, 16, 32, 64]
FEAT_SHAPES = [(64, 64), (32, 32), (16, 16), (8, 8)]
CENTER_RADIUS = 2.5
CAND_TOPK = 10
IOU_W = 3.0
CLS_W = 1.0
VFL_ALPHA = 0.75
VFL_GAMMA = 2.0
W_VFL = 1.0
W_GIOU = 2.0
W_DFL = 0.25

N_PRIORS = sum(h * w for h, w in FEAT_SHAPES)
BIGI = 10**9


def _priors():
    out = []
    for (h, w), s in zip(FEAT_SHAPES, STRIDES):
        ys = (jnp.arange(h, dtype=jnp.float32) + 0.5) * s
        xs = (jnp.arange(w, dtype=jnp.float32) + 0.5) * s
        yy, xx = jnp.meshgrid(ys, xs, indexing='ij')
        cx = xx.reshape(-1)
        cy = yy.reshape(-1)
        ss = jnp.full_like(cx, float(s))
        out.append(jnp.stack([cx, cy, ss, ss], axis=-1))
    return jnp.concatenate(out, axis=0)


def _body(cls_ref, reg_ref, gtb_ref, gtl_ref, pri_ref, out_ref):
    N = N_PRIORS
    M = gtl_ref.shape[1]
    f32 = jnp.float32

    cls = cls_ref[0]
    reg = reg_ref[0]
    gtb = gtb_ref[0]
    gtl = gtl_ref[0]
    cx = pri_ref[0:1, :]
    cy = pri_ref[1:2, :]
    st = pri_ref[2:3, :]

    colmax = jnp.max(reg, axis=0, keepdims=True)
    e_in = reg - colmax
    ex = jnp.exp(e_in)
    sub32 = jax.lax.broadcasted_iota(jnp.int32, (32, N), 0)
    grp = sub32 // 8
    j8 = (sub32 % 8).astype(f32)
    dists = []
    ldens = []
    for g in range(4):
        m = grp == g
        den = jnp.sum(jnp.where(m, ex, 0.0), axis=0, keepdims=True)
        num = jnp.sum(jnp.where(m, ex * j8, 0.0), axis=0, keepdims=True)
        dists.append(num / den * st)
        ldens.append(jnp.log(den))
    x1 = cx - dists[0]
    y1 = cy - dists[1]
    x2 = cx + dists[2]
    y2 = cy + dists[3]

    ecls = jnp.exp(cls)
    p = ecls / (1.0 + ecls)
    sp = jnp.log(1.0 + ecls)
    l1 = jnp.maximum(cls - sp, jnp.log(jnp.float32(1e-12)))
    l0 = jnp.maximum(-sp, jnp.log(jnp.float32(1e-12)))
    log_lo = jnp.log(jnp.float32(1e-7))
    log_hi = jnp.log(jnp.float32(1.0 - 1e-7))
    l1c = jnp.clip(l1, log_lo, log_hi)
    l0c = jnp.clip(l0, log_lo, log_hi)
    S = -jnp.sum(l0c, axis=0, keepdims=True)
    Acost = l0c - l1c
    oh_mc = (jax.lax.broadcasted_iota(jnp.int32, (M, NUM_CLASSES), 1)
             == gtl).astype(f32)
    gsel = jax.lax.dot_general(oh_mc, Acost, (((1,), (0,)), ((), ())),
                               preferred_element_type=f32,
                               precision=jax.lax.Precision.HIGHEST)
    cls_cost = S + gsel

    gx1 = gtb[:, 0:1]
    gy1 = gtb[:, 1:2]
    gx2 = gtb[:, 2:3]
    gy2 = gtb[:, 3:4]
    l_ = cx - gx1
    t_ = cy - gy1
    r_ = gx2 - cx
    b_ = gy2 - cy
    gmin = jnp.minimum(jnp.minimum(l_, t_), jnp.minimum(r_, b_))
    gcx = (gx1 + gx2) * 0.5
    gcy = (gy1 + gy2) * 0.5
    cl_ = cx - (gcx - CENTER_RADIUS * st)
    ct_ = cy - (gcy - CENTER_RADIUS * st)
    cr_ = (gcx + CENTER_RADIUS * st) - cx
    cb_ = (gcy + CENTER_RADIUS * st) - cy
    cmin = jnp.minimum(jnp.minimum(cl_, ct_), jnp.minimum(cr_, cb_))
    bothv = jnp.minimum(gmin, cmin)
    validv = jnp.max(jnp.maximum(gmin, cmin), axis=0, keepdims=True)
    validf = (validv > 0).astype(f32)

    area_p = (x2 - x1) * (y2 - y1)
    area_g = (gx2 - gx1) * (gy2 - gy1)
    tlx = jnp.maximum(x1, gx1)
    tly = jnp.maximum(y1, gy1)
    brx = jnp.minimum(x2, gx2)
    bry = jnp.minimum(y2, gy2)
    iw = jnp.clip(brx - tlx, 0.0, None)
    ih = jnp.clip(bry - tly, 0.0, None)
    inter = iw * ih
    ious = inter / (area_p + area_g - inter + 1e-16) * validf
    iou_cost = -jnp.log(ious + 1e-8)
    cost = (CLS_W * cls_cost + IOU_W * iou_cost
            + jnp.where(bothv > 0, 0.0, 1e5) + 1e5 * (1.0 - validf))

    iotaL = jax.lax.broadcasted_iota(jnp.int32, (M, N), 1)
    iotaS = jax.lax.broadcasted_iota(jnp.int32, (M, N), 0)

    iou_work = ious
    topk_sum = jnp.zeros((M, 1), f32)
    for _ in range(CAND_TOPK):
        mval = jnp.max(iou_work, axis=1, keepdims=True)
        topk_sum = topk_sum + mval
        idx = jnp.argmax(iou_work, axis=1, keepdims=True)
        iou_work = jnp.where(iotaL == idx, -1.0, iou_work)
    dyn_k = jnp.maximum(topk_sum.astype(jnp.int32), 1)

    cost_work = cost
    matchf = jnp.zeros((M, N), f32)
    for j in range(CAND_TOPK):
        idx = jnp.argmin(cost_work, axis=1, keepdims=True)
        jltf = (j < dyn_k).astype(f32)
        hit = iotaL == idx
        matchf = jnp.where(hit, jltf, matchf)
        cost_work = jnp.where(hit, 1e30, cost_work)
    matchingf = matchf * validf

    cnt = jnp.sum(matchingf, axis=0, keepdims=True)
    masked_cost = jnp.where(matchingf > 0, cost, 1e30)
    mgt = jnp.argmin(masked_cost, axis=0, keepdims=True)
    fg = (cnt > 0).astype(f32)
    oh_m = iotaS == mgt
    miou = jnp.sum(jnp.where(oh_m, ious, 0.0), axis=0, keepdims=True) * fg

    glab = jnp.sum(jnp.where(oh_m, gtl, 0), axis=0, keepdims=True)
    qscale = miou * fg
    ohc = jax.lax.broadcasted_iota(jnp.int32, (NUM_CLASSES, N), 0) == glab
    q = jnp.where(ohc, qscale, 0.0)
    fw = jnp.where(q > 0, q, VFL_ALPHA * p * p)
    bce = -(q * l1 + (1.0 - q) * l0)
    vfl = jnp.sum(bce * fw)

    tx1 = jnp.sum(jnp.where(oh_m, gx1, 0.0), axis=0, keepdims=True)
    ty1 = jnp.sum(jnp.where(oh_m, gy1, 0.0), axis=0, keepdims=True)
    tx2 = jnp.sum(jnp.where(oh_m, gx2, 0.0), axis=0, keepdims=True)
    ty2 = jnp.sum(jnp.where(oh_m, gy2, 0.0), axis=0, keepdims=True)
    w = miou * fg
    ttlx = jnp.maximum(x1, tx1)
    ttly = jnp.maximum(y1, ty1)
    tbrx = jnp.minimum(x2, tx2)
    tbry = jnp.minimum(y2, ty2)
    giw = jnp.clip(tbrx - ttlx, 0.0, None)
    gih = jnp.clip(tbry - ttly, 0.0, None)
    ginter = giw * gih
    ap = jnp.clip(x2 - x1, 0.0, None) * jnp.clip(y2 - y1, 0.0, None)
    at = jnp.clip(tx2 - tx1, 0.0, None) * jnp.clip(ty2 - ty1, 0.0, None)
    union = ap + at - ginter
    giou_iou = ginter / (union + 1e-16)
    ctlx = jnp.minimum(x1, tx1)
    ctly = jnp.minimum(y1, ty1)
    cbrx = jnp.maximum(x2, tx2)
    cbry = jnp.maximum(y2, ty2)
    cw = jnp.clip(cbrx - ctlx, 0.0, None)
    ch = jnp.clip(cbry - ctly, 0.0, None)
    ac = cw * ch
    giou_l = 1.0 - (giou_iou - (ac - union) / (ac + 1e-16))
    gl = jnp.sum(giou_l * w)

    left = (cx - tx1) / st
    top = (cy - ty1) / st
    right = (tx2 - cx) / st
    bot = (ty2 - cy) / st
    td = jnp.where(grp == 0, left,
                   jnp.where(grp == 1, top,
                             jnp.where(grp == 2, right, bot)))
    td = jnp.clip(td, 0.0, REG_MAX - 0.1)
    dlf = jnp.floor(td)
    drf = jnp.minimum(dlf + 1.0, float(REG_MAX))
    wl = dlf + 1.0 - td
    wr = td - dlf
    ld = jnp.where(grp == 0, ldens[0],
                   jnp.where(grp == 1, ldens[1],
                             jnp.where(grp == 2, ldens[2], ldens[3])))
    logp = e_in - ld
    Wt = wl * (j8 == dlf).astype(f32) + wr * (j8 == drf).astype(f32)
    ce_sum = jnp.sum(-logp * Wt, axis=0, keepdims=True)
    dfl = jnp.sum((ce_sum / 4.0) * w)

    num_pos = jnp.sum(fg)
    wsum = jnp.sum(w)

    lane = jax.lax.broadcasted_iota(jnp.int32, (1, 128), 1)
    row = (jnp.where(lane == 0, vfl, 0.0) + jnp.where(lane == 1, gl, 0.0)
           + jnp.where(lane == 2, dfl, 0.0) + jnp.where(lane == 3, num_pos, 0.0)
           + jnp.where(lane == 4, wsum, 0.0))
    out_ref[0] = row


@jax.jit
def kernel(cls_scores, reg_preds, gt_bboxes, gt_labels):
    B, N, C = cls_scores.shape
    M = gt_labels.shape[1]
    pri_t = jnp.transpose(_priors())
    cls_t = jnp.transpose(cls_scores, (0, 2, 1))
    reg_t = jnp.transpose(reg_preds, (0, 2, 1))
    gtl3 = gt_labels.reshape(B, M, 1).astype(jnp.int32)
    comps = pl.pallas_call(
        _body,
        grid=(B,),
        in_specs=[
            pl.BlockSpec((1, C, N), lambda b: (b, 0, 0)),
            pl.BlockSpec((1, 4 * (REG_MAX + 1), N), lambda b: (b, 0, 0)),
            pl.BlockSpec((1, M, 4), lambda b: (b, 0, 0)),
            pl.BlockSpec((1, M, 1), lambda b: (b, 0, 0)),
            pl.BlockSpec((4, N), lambda b: (0, 0)),
        ],
        out_specs=pl.BlockSpec((1, 1, 128), lambda b: (b, 0, 0)),
        out_shape=jax.ShapeDtypeStruct((B, 1, 128), jnp.float32),
        compiler_params=pltpu.CompilerParams(
            vmem_limit_bytes=100 * 1024 * 1024),
    )(cls_t, reg_t, gt_bboxes, gtl3, pri_t)
    vfl = comps[:, 0, 0]
    gl = comps[:, 0, 1]
    dfl = comps[:, 0, 2]
    num_pos = jnp.maximum(comps[:, 0, 3], 1.0)
    wsum = jnp.maximum(comps[:, 0, 4], 1.0)
    per_b = W_VFL * vfl / num_pos + W_GIOU * gl / wsum + W_DFL * dfl / wsum
    return jnp.mean(per_b)


# speedup vs baseline: 1.1062x; 1.0004x over previous
import jax
import jax.numpy as jnp
from jax.experimental import pallas as pl
from jax.experimental.pallas import tpu as pltpu

NUM_CLASSES = 80
REG_MAX = 7
STRIDES = [8, 16, 32, 64]
FEAT_SHAPES = [(64, 64), (32, 32), (16, 16), (8, 8)]
CENTER_RADIUS = 2.5
CAND_TOPK = 10
IOU_W = 3.0
CLS_W = 1.0
VFL_ALPHA = 0.75
VFL_GAMMA = 2.0
W_VFL = 1.0
W_GIOU = 2.0
W_DFL = 0.25

N_PRIORS = sum(h * w for h, w in FEAT_SHAPES)


def _priors():
    out = []
    for (h, w), s in zip(FEAT_SHAPES, STRIDES):
        ys = (jnp.arange(h, dtype=jnp.float32) + 0.5) * s
        xs = (jnp.arange(w, dtype=jnp.float32) + 0.5) * s
        yy, xx = jnp.meshgrid(ys, xs, indexing='ij')
        cx = xx.reshape(-1)
        cy = yy.reshape(-1)
        ss = jnp.full_like(cx, float(s))
        out.append(jnp.stack([cx, cy, ss, ss], axis=-1))
    return jnp.concatenate(out, axis=0)


def _body(cls_ref, reg_ref, gtb_ref, gtl_ref, pri_ref, out_ref):
    N = N_PRIORS
    M = gtl_ref.shape[1]
    f32 = jnp.float32

    cls = cls_ref[0]
    reg = reg_ref[0]
    gtb = gtb_ref[0]
    gtl = gtl_ref[0]
    cx = pri_ref[0:1, :]
    cy = pri_ref[1:2, :]
    st = pri_ref[2:3, :]

    colmax = jnp.max(reg, axis=0, keepdims=True)
    e_in = reg - colmax
    ex = jnp.exp(e_in)
    sub32 = jax.lax.broadcasted_iota(jnp.int32, (32, N), 0)
    grp = sub32 // 8
    j8 = (sub32 % 8).astype(f32)
    dists = []
    ldens = []
    for g in range(4):
        m = grp == g
        den = jnp.sum(jnp.where(m, ex, 0.0), axis=0, keepdims=True)
        num = jnp.sum(jnp.where(m, ex * j8, 0.0), axis=0, keepdims=True)
        dists.append(num / den * st)
        ldens.append(jnp.log(den))
    x1 = cx - dists[0]
    y1 = cy - dists[1]
    x2 = cx + dists[2]
    y2 = cy + dists[3]

    ecls = jnp.exp(cls)
    p = ecls / (1.0 + ecls)
    sp = jnp.log(1.0 + ecls)
    l1 = jnp.maximum(cls - sp, jnp.log(jnp.float32(1e-12)))
    l0 = jnp.maximum(-sp, jnp.log(jnp.float32(1e-12)))
    log_lo = jnp.log(jnp.float32(1e-7))
    log_hi = jnp.log(jnp.float32(1.0 - 1e-7))
    l1c = jnp.clip(l1, log_lo, log_hi)
    l0c = jnp.clip(l0, log_lo, log_hi)
    S = -jnp.sum(l0c, axis=0, keepdims=True)
    Acost = l0c - l1c
    oh_mc = (jax.lax.broadcasted_iota(jnp.int32, (M, NUM_CLASSES), 1)
             == gtl).astype(f32)
    gsel = jax.lax.dot_general(oh_mc, Acost, (((1,), (0,)), ((), ())),
                               preferred_element_type=f32,
                               precision=jax.lax.Precision.HIGHEST)
    cls_cost = S + gsel

    gx1 = gtb[:, 0:1]
    gy1 = gtb[:, 1:2]
    gx2 = gtb[:, 2:3]
    gy2 = gtb[:, 3:4]
    l_ = cx - gx1
    t_ = cy - gy1
    r_ = gx2 - cx
    b_ = gy2 - cy
    gmin = jnp.minimum(jnp.minimum(l_, t_), jnp.minimum(r_, b_))
    gcx = (gx1 + gx2) * 0.5
    gcy = (gy1 + gy2) * 0.5
    cl_ = cx - (gcx - CENTER_RADIUS * st)
    ct_ = cy - (gcy - CENTER_RADIUS * st)
    cr_ = (gcx + CENTER_RADIUS * st) - cx
    cb_ = (gcy + CENTER_RADIUS * st) - cy
    cmin = jnp.minimum(jnp.minimum(cl_, ct_), jnp.minimum(cr_, cb_))
    bothv = jnp.minimum(gmin, cmin)
    validv = jnp.max(jnp.maximum(gmin, cmin), axis=0, keepdims=True)
    validf = (validv > 0).astype(f32)

    area_p = (x2 - x1) * (y2 - y1)
    area_g = (gx2 - gx1) * (gy2 - gy1)
    tlx = jnp.maximum(x1, gx1)
    tly = jnp.maximum(y1, gy1)
    brx = jnp.minimum(x2, gx2)
    bry = jnp.minimum(y2, gy2)
    iw = jnp.clip(brx - tlx, 0.0, None)
    ih = jnp.clip(bry - tly, 0.0, None)
    inter = iw * ih
    ious = inter / (area_p + area_g - inter + 1e-16) * validf
    iou_cost = -jnp.log(ious + 1e-8)
    cost = (CLS_W * cls_cost + IOU_W * iou_cost
            + jnp.where(bothv > 0, 0.0, 1e5) + 1e5 * (1.0 - validf))

    iotaL = jax.lax.broadcasted_iota(jnp.int32, (M, N), 1)
    iotaS = jax.lax.broadcasted_iota(jnp.int32, (M, N), 0)

    iou_work = ious
    topk_sum = jnp.zeros((M, 1), f32)
    for _ in range(CAND_TOPK):
        mval = jnp.max(iou_work, axis=1, keepdims=True)
        topk_sum = topk_sum + mval
        idx = jnp.argmax(iou_work, axis=1, keepdims=True)
        iou_work = jnp.where(iotaL == idx, -1.0, iou_work)
    dyn_k = jnp.maximum(topk_sum.astype(jnp.int32), 1)

    cost_work = cost
    matchf = jnp.zeros((M, N), f32)
    for j in range(CAND_TOPK):
        idx = jnp.argmin(cost_work, axis=1, keepdims=True)
        jltf = (j < dyn_k).astype(f32)
        hit = iotaL == idx
        matchf = jnp.where(hit, jltf, matchf)
        cost_work = jnp.where(hit, 1e30, cost_work)
    matchingf = matchf * validf

    cnt = jnp.sum(matchingf, axis=0, keepdims=True)
    masked_cost = jnp.where(matchingf > 0, cost, 1e30)
    mgt = jnp.argmin(masked_cost, axis=0, keepdims=True)
    fg = (cnt > 0).astype(f32)
    oh_m = iotaS == mgt
    miou = jnp.sum(jnp.where(oh_m, ious, 0.0), axis=0, keepdims=True) * fg

    glab = jnp.sum(jnp.where(oh_m, gtl, 0), axis=0, keepdims=True)
    qscale = miou * fg
    ohc = jax.lax.broadcasted_iota(jnp.int32, (NUM_CLASSES, N), 0) == glab
    q = jnp.where(ohc, qscale, 0.0)
    fw = jnp.where(q > 0, q, VFL_ALPHA * p * p)
    bce = -(q * l1 + (1.0 - q) * l0)
    vfl = jnp.sum(bce * fw)

    tx1 = jnp.sum(jnp.where(oh_m, gx1, 0.0), axis=0, keepdims=True)
    ty1 = jnp.sum(jnp.where(oh_m, gy1, 0.0), axis=0, keepdims=True)
    tx2 = jnp.sum(jnp.where(oh_m, gx2, 0.0), axis=0, keepdims=True)
    ty2 = jnp.sum(jnp.where(oh_m, gy2, 0.0), axis=0, keepdims=True)
    w = miou * fg
    ttlx = jnp.maximum(x1, tx1)
    ttly = jnp.maximum(y1, ty1)
    tbrx = jnp.minimum(x2, tx2)
    tbry = jnp.minimum(y2, ty2)
    giw = jnp.clip(tbrx - ttlx, 0.0, None)
    gih = jnp.clip(tbry - ttly, 0.0, None)
    ginter = giw * gih
    ap = jnp.clip(x2 - x1, 0.0, None) * jnp.clip(y2 - y1, 0.0, None)
    at = jnp.clip(tx2 - tx1, 0.0, None) * jnp.clip(ty2 - ty1, 0.0, None)
    union = ap + at - ginter
    giou_iou = ginter / (union + 1e-16)
    ctlx = jnp.minimum(x1, tx1)
    ctly = jnp.minimum(y1, ty1)
    cbrx = jnp.maximum(x2, tx2)
    cbry = jnp.maximum(y2, ty2)
    cw = jnp.clip(cbrx - ctlx, 0.0, None)
    ch = jnp.clip(cbry - ctly, 0.0, None)
    ac = cw * ch
    giou_l = 1.0 - (giou_iou - (ac - union) / (ac + 1e-16))
    gl = jnp.sum(giou_l * w)

    left = (cx - tx1) / st
    top = (cy - ty1) / st
    right = (tx2 - cx) / st
    bot = (ty2 - cy) / st
    td = jnp.where(grp == 0, left,
                   jnp.where(grp == 1, top,
                             jnp.where(grp == 2, right, bot)))
    td = jnp.clip(td, 0.0, REG_MAX - 0.1)
    dlf = jnp.floor(td)
    drf = jnp.minimum(dlf + 1.0, float(REG_MAX))
    wl = dlf + 1.0 - td
    wr = td - dlf
    ld = jnp.where(grp == 0, ldens[0],
                   jnp.where(grp == 1, ldens[1],
                             jnp.where(grp == 2, ldens[2], ldens[3])))
    logp = e_in - ld
    Wt = wl * (j8 == dlf).astype(f32) + wr * (j8 == drf).astype(f32)
    ce_sum = jnp.sum(-logp * Wt, axis=0, keepdims=True)
    dfl = jnp.sum((ce_sum / 4.0) * w)

    num_pos = jnp.sum(fg)
    wsum = jnp.sum(w)

    lane = jax.lax.broadcasted_iota(jnp.int32, (1, 128), 1)
    row = (jnp.where(lane == 0, vfl, 0.0) + jnp.where(lane == 1, gl, 0.0)
           + jnp.where(lane == 2, dfl, 0.0) + jnp.where(lane == 3, num_pos, 0.0)
           + jnp.where(lane == 4, wsum, 0.0))
    out_ref[0] = row


@jax.jit
def kernel(cls_scores, reg_preds, gt_bboxes, gt_labels):
    B, N, C = cls_scores.shape
    M = gt_labels.shape[1]
    pri_t = jnp.transpose(_priors())
    cls_t = jnp.transpose(cls_scores, (0, 2, 1))
    reg_t = jnp.transpose(reg_preds, (0, 2, 1))
    gtl3 = gt_labels.reshape(B, M, 1).astype(jnp.int32)
    comps = pl.pallas_call(
        _body,
        grid=(B,),
        in_specs=[
            pl.BlockSpec((1, C, N), lambda b: (b, 0, 0)),
            pl.BlockSpec((1, 4 * (REG_MAX + 1), N), lambda b: (b, 0, 0)),
            pl.BlockSpec((1, M, 4), lambda b: (b, 0, 0)),
            pl.BlockSpec((1, M, 1), lambda b: (b, 0, 0)),
            pl.BlockSpec((4, N), lambda b: (0, 0)),
        ],
        out_specs=pl.BlockSpec((1, 1, 128), lambda b: (b, 0, 0)),
        out_shape=jax.ShapeDtypeStruct((B, 1, 128), jnp.float32),
        compiler_params=pltpu.CompilerParams(
            vmem_limit_bytes=100 * 1024 * 1024),
    )(cls_t, reg_t, gt_bboxes, gtl3, pri_t)
    vfl = comps[:, 0, 0]
    gl = comps[:, 0, 1]
    dfl = comps[:, 0, 2]
    num_pos = jnp.maximum(comps[:, 0, 3], 1.0)
    wsum = jnp.maximum(comps[:, 0, 4], 1.0)
    per_b = W_VFL * vfl / num_pos + W_GIOU * gl / wsum + W_DFL * dfl / wsum
    return jnp.mean(per_b)
